# baseline (speedup 1.0000x reference)
import functools

import jax
import jax.numpy as jnp
from jax import lax
from jax.experimental import pallas as pl
from jax.experimental.pallas import tpu as pltpu
from jax.experimental.pallas import tpu_sc as plsc

N_ITEMS = 1000000
N_RANKS = 20
BATCH = 16384
TOTAL = BATCH * N_RANKS

_info = plsc.get_sparse_core_info()
NC = _info.num_cores
NS = _info.num_subcores
NW = NC * NS
L = 16

PER_W = TOTAL // NW
ROWS = BATCH // NW
NCH = 16
NBUF = 4
CROWS = ROWS // NCH
CVALS = PER_W // NCH
UNROLL = 4

_mesh = plsc.VectorSubcoreMesh(core_axis_name="c", subcore_axis_name="s")


def _sigmoid(v):
    return 1.0 / (1.0 + jnp.exp(-v))


@functools.partial(
    pl.kernel,
    mesh=_mesh,
    out_type=jax.ShapeDtypeStruct((BATCH, N_RANKS), jnp.float32),
    scratch_types=[
        pltpu.VMEM((PER_W,), jnp.int32),
        pltpu.VMEM((CVALS + L,), jnp.float32),
        pltpu.VMEM((CVALS + L,), jnp.float32),
        pltpu.VMEM((CVALS + L,), jnp.float32),
        pltpu.VMEM((CVALS + L,), jnp.float32),
        pltpu.VMEM((ROWS, N_RANKS), jnp.float32),
        pltpu.VMEM((32,), jnp.float32),
        pltpu.SemaphoreType.DMA,
        pltpu.SemaphoreType.DMA,
        pltpu.SemaphoreType.DMA,
        pltpu.SemaphoreType.DMA,
        pltpu.SemaphoreType.DMA,
        pltpu.SemaphoreType.DMA,
    ],
)
def _pbm_kernel(x_hbm, exam_hbm, rel_hbm, out_hbm,
                idx_v, vals_a, vals_b, vals_c, vals_d, resv, exam_v,
                gsem_a, gsem_b, gsem_c, gsem_d, isem, osem):
    wid = lax.axis_index("s") * NC + lax.axis_index("c")
    base = wid * PER_W
    base_row = wid * ROWS

    vals = (vals_a, vals_b, vals_c, vals_d)
    gsem = (gsem_a, gsem_b, gsem_c, gsem_d)

    idx_copies = [
        pltpu.async_copy(
            x_hbm.at[pl.ds(base + c * CVALS, CVALS)],
            idx_v.at[pl.ds(c * CVALS, CVALS)],
            isem,
        )
        for c in range(NCH)
    ]

    pltpu.sync_copy(exam_hbm, exam_v)
    e0 = _sigmoid(exam_v[pl.ds(0, L)])
    e1 = _sigmoid(exam_v[pl.ds(4, L)])

    def start_gather(c):
        idx_copies[c].wait()
        return pltpu.async_copy(
            rel_hbm.at[idx_v.at[pl.ds(c * CVALS, CVALS)]],
            vals[c % NBUF].at[pl.ds(0, CVALS)],
            gsem[c % NBUF],
        )

    gathers = [start_gather(c) for c in range(NBUF - 1)]
    out_copies = []
    for c in range(NCH):
        if c + NBUF - 1 < NCH:
            gathers.append(start_gather(c + NBUF - 1))
        gathers[c].wait()
        buf = vals[c % NBUF]

        def group_body(g, _, buf=buf, c=c):
            for u in range(UNROLL):
                r = g * UNROLL + u
                b = r * N_RANKS
                v0 = buf[pl.ds(b, L)]
                v1 = buf[pl.ds(b + 4, L)]
                row = c * CROWS + r
                resv[row, pl.ds(0, L)] = e0 * _sigmoid(v0)
                resv[row, pl.ds(4, L)] = e1 * _sigmoid(v1)
            return 0

        lax.fori_loop(0, CROWS // UNROLL, group_body, 0)
        out_copies.append(
            pltpu.async_copy(
                resv.at[pl.ds(c * CROWS, CROWS)],
                out_hbm.at[pl.ds(base_row + c * CROWS, CROWS)],
                osem,
            )
        )

    for oc in out_copies:
        oc.wait()


def kernel(x, exam_table, rel_table):
    xf = x.reshape(TOTAL)
    exam = jnp.pad(exam_table.reshape(N_RANKS), (0, 32 - N_RANKS))
    rel = rel_table.reshape(N_ITEMS)
    return _pbm_kernel(xf, exam, rel)


# speedup vs baseline: 1.3056x; 1.3056x over previous
import functools

import jax
import jax.numpy as jnp
from jax import lax
from jax.experimental import pallas as pl
from jax.experimental.pallas import tpu as pltpu
from jax.experimental.pallas import tpu_sc as plsc

N_ITEMS = 1000000
N_RANKS = 20
BATCH = 16384
TOTAL = BATCH * N_RANKS

_info = plsc.get_sparse_core_info()
NC = _info.num_cores
NS = _info.num_subcores
NW = NC * NS
L = 16

COLS = BATCH // NW
NBUF = 4
UNROLL = 4

_mesh = plsc.VectorSubcoreMesh(core_axis_name="c", subcore_axis_name="s")


def _sigmoid(v):
    return 1.0 / (1.0 + jnp.exp(-v))


@functools.partial(
    pl.kernel,
    mesh=_mesh,
    out_type=jax.ShapeDtypeStruct((N_RANKS, BATCH), jnp.float32),
    scratch_types=[
        pltpu.VMEM((N_RANKS * COLS,), jnp.int32),
        pltpu.VMEM((N_RANKS * COLS,), jnp.float32),
        pltpu.VMEM((N_RANKS * COLS,), jnp.float32),
        pltpu.VMEM((48,), jnp.float32),
        pltpu.SemaphoreType.DMA,
        pltpu.SemaphoreType.DMA,
        pltpu.SemaphoreType.DMA,
        pltpu.SemaphoreType.DMA,
        pltpu.SemaphoreType.DMA,
        pltpu.SemaphoreType.DMA,
    ],
)
def _pbm_kernel(xt_hbm, exam_hbm, rel_hbm, out_hbm,
                idx_v, vals_v, resv, exam_v,
                gsem_a, gsem_b, gsem_c, gsem_d, isem, osem):
    wid = lax.axis_index("s") * NC + lax.axis_index("c")
    col0 = wid * COLS

    gsem = (gsem_a, gsem_b, gsem_c, gsem_d)

    idx_copies = [
        pltpu.async_copy(
            xt_hbm.at[r, pl.ds(col0, COLS)],
            idx_v.at[pl.ds(r * COLS, COLS)],
            isem,
        )
        for r in range(N_RANKS)
    ]
    pltpu.sync_copy(exam_hbm, exam_v)

    def start_gather(r):
        idx_copies[r].wait()
        return pltpu.async_copy(
            rel_hbm.at[idx_v.at[pl.ds(r * COLS, COLS)]],
            vals_v.at[pl.ds(r * COLS, COLS)],
            gsem[r % NBUF],
        )

    gathers = [start_gather(r) for r in range(NBUF - 1)]
    out_copies = []
    for r in range(N_RANKS):
        if r + NBUF - 1 < N_RANKS:
            gathers.append(start_gather(r + NBUF - 1))
        gathers[r].wait()
        ev = exam_v[pl.ds(r, L)]
        er = _sigmoid(jnp.full((L,), ev[0], jnp.float32))

        def group_body(g, _, r=r, er=er):
            for u in range(UNROLL):
                b = r * COLS + (g * UNROLL + u) * L
                resv[pl.ds(b, L)] = er * _sigmoid(vals_v[pl.ds(b, L)])
            return 0

        lax.fori_loop(0, COLS // (UNROLL * L), group_body, 0)
        out_copies.append(
            pltpu.async_copy(
                resv.at[pl.ds(r * COLS, COLS)],
                out_hbm.at[r, pl.ds(col0, COLS)],
                osem,
            )
        )

    for oc in out_copies:
        oc.wait()


def kernel(x, exam_table, rel_table):
    xt = x.T
    exam = jnp.pad(exam_table.reshape(N_RANKS), (0, 48 - N_RANKS))
    rel = rel_table.reshape(N_ITEMS)
    out_t = _pbm_kernel(xt, exam, rel)
    return out_t.T


# speedup vs baseline: 2.7031x; 2.0704x over previous
import functools

import jax
import jax.numpy as jnp
from jax import lax
from jax.experimental import pallas as pl
from jax.experimental.pallas import tpu as pltpu
from jax.experimental.pallas import tpu_sc as plsc

N_ITEMS = 1000000
N_RANKS = 20
BATCH = 16384
TOTAL = BATCH * N_RANKS

_info = plsc.get_sparse_core_info()
NC = _info.num_cores
NS = _info.num_subcores
NW = NC * NS
L = 16

COLS = BATCH // NW
NBUF = 4
UNROLL = 4

_mesh = plsc.VectorSubcoreMesh(core_axis_name="c", subcore_axis_name="s")


def _sigmoid(v):
    return 1.0 / (1.0 + jnp.exp(-v))


@functools.partial(
    pl.kernel,
    mesh=_mesh,
    out_type=jax.ShapeDtypeStruct((N_RANKS, BATCH), jnp.float32),
    scratch_types=[
        pltpu.VMEM((N_RANKS * COLS,), jnp.int32),
        pltpu.VMEM((N_RANKS * COLS,), jnp.float32),
        pltpu.VMEM((N_RANKS * COLS,), jnp.float32),
        pltpu.VMEM((48,), jnp.float32),
        pltpu.SemaphoreType.DMA,
        pltpu.SemaphoreType.DMA,
        pltpu.SemaphoreType.DMA,
        pltpu.SemaphoreType.DMA,
        pltpu.SemaphoreType.DMA,
        pltpu.SemaphoreType.DMA,
    ],
)
def _pbm_kernel(xt_hbm, exam_hbm, rel_hbm, out_hbm,
                idx_v, vals_v, resv, exam_v,
                gsem_a, gsem_b, gsem_c, gsem_d, isem, osem):
    wid = lax.axis_index("s") * NC + lax.axis_index("c")
    col0 = wid * COLS

    gsem = (gsem_a, gsem_b, gsem_c, gsem_d)
    rel_row = rel_hbm.at[0]

    idx_copies = [
        pltpu.async_copy(
            xt_hbm.at[r, pl.ds(col0, COLS)],
            idx_v.at[pl.ds(r * COLS, COLS)],
            isem,
        )
        for r in range(N_RANKS)
    ]
    pltpu.sync_copy(exam_hbm, exam_v)

    def start_gather(r):
        idx_copies[r].wait()
        return pltpu.async_copy(
            rel_row.at[idx_v.at[pl.ds(r * COLS, COLS)]],
            vals_v.at[pl.ds(r * COLS, COLS)],
            gsem[r % NBUF],
        )

    gathers = [start_gather(r) for r in range(NBUF - 1)]
    out_copies = []
    for r in range(N_RANKS):
        if r + NBUF - 1 < N_RANKS:
            gathers.append(start_gather(r + NBUF - 1))
        gathers[r].wait()
        ev = exam_v[pl.ds(r, L)]
        er = _sigmoid(jnp.full((L,), ev[0], jnp.float32))

        def group_body(g, _, r=r, er=er):
            for u in range(UNROLL):
                b = r * COLS + (g * UNROLL + u) * L
                resv[pl.ds(b, L)] = er * _sigmoid(vals_v[pl.ds(b, L)])
            return 0

        lax.fori_loop(0, COLS // (UNROLL * L), group_body, 0)
        out_copies.append(
            pltpu.async_copy(
                resv.at[pl.ds(r * COLS, COLS)],
                out_hbm.at[r, pl.ds(col0, COLS)],
                osem,
            )
        )

    for oc in out_copies:
        oc.wait()


def kernel(x, exam_table, rel_table):
    xt = x.T
    exam = jnp.pad(exam_table.reshape(N_RANKS), (0, 48 - N_RANKS))
    rel = rel_table.T
    out_t = _pbm_kernel(xt, exam, rel)
    return out_t.T


# speedup vs baseline: 2.7392x; 1.0134x over previous
import functools

import jax
import jax.numpy as jnp
from jax import lax
from jax.experimental import pallas as pl
from jax.experimental.pallas import tpu as pltpu
from jax.experimental.pallas import tpu_sc as plsc

N_ITEMS = 1000000
N_RANKS = 20
BATCH = 16384
TOTAL = BATCH * N_RANKS

_info = plsc.get_sparse_core_info()
NC = _info.num_cores
NS = _info.num_subcores
NW = NC * NS
L = 16

COLS = BATCH // NW
NBUF = 6
UNROLL = 4

_mesh = plsc.VectorSubcoreMesh(core_axis_name="c", subcore_axis_name="s")


def _sigmoid(v):
    return 1.0 / (1.0 + jnp.exp(-v))


@functools.partial(
    pl.kernel,
    mesh=_mesh,
    out_type=jax.ShapeDtypeStruct((N_RANKS, BATCH), jnp.float32),
    scratch_types=[
        pltpu.VMEM((N_RANKS * COLS,), jnp.int32),
        pltpu.VMEM((N_RANKS * COLS,), jnp.float32),
        pltpu.VMEM((N_RANKS * COLS,), jnp.float32),
        pltpu.VMEM((48,), jnp.float32),
        pltpu.SemaphoreType.DMA,
        pltpu.SemaphoreType.DMA,
        pltpu.SemaphoreType.DMA,
        pltpu.SemaphoreType.DMA,
        pltpu.SemaphoreType.DMA,
        pltpu.SemaphoreType.DMA,
        pltpu.SemaphoreType.DMA,
        pltpu.SemaphoreType.DMA,
    ],
)
def _pbm_kernel(xt_hbm, exam_hbm, rel_hbm, out_hbm,
                idx_v, vals_v, resv, exam_v,
                gsem_a, gsem_b, gsem_c, gsem_d, gsem_e, gsem_f, isem, osem):
    wid = lax.axis_index("s") * NC + lax.axis_index("c")
    col0 = wid * COLS

    gsem = (gsem_a, gsem_b, gsem_c, gsem_d, gsem_e, gsem_f)
    rel_row = rel_hbm.at[0]

    idx_copies = [
        pltpu.async_copy(
            xt_hbm.at[r, pl.ds(col0, COLS)],
            idx_v.at[pl.ds(r * COLS, COLS)],
            isem,
        )
        for r in range(N_RANKS)
    ]

    def start_gather(r):
        idx_copies[r].wait()
        return pltpu.async_copy(
            rel_row.at[idx_v.at[pl.ds(r * COLS, COLS)]],
            vals_v.at[pl.ds(r * COLS, COLS)],
            gsem[r % NBUF],
        )

    gathers = [start_gather(r) for r in range(NBUF - 1)]
    pltpu.sync_copy(exam_hbm.at[0], exam_v.at[pl.ds(0, N_RANKS)])
    out_copies = []
    for r in range(N_RANKS):
        if r + NBUF - 1 < N_RANKS:
            gathers.append(start_gather(r + NBUF - 1))
        gathers[r].wait()
        ev = exam_v[pl.ds(r, L)]
        er = _sigmoid(jnp.full((L,), ev[0], jnp.float32))

        def group_body(g, _, r=r, er=er):
            for u in range(UNROLL):
                b = r * COLS + (g * UNROLL + u) * L
                resv[pl.ds(b, L)] = er * _sigmoid(vals_v[pl.ds(b, L)])
            return 0

        lax.fori_loop(0, COLS // (UNROLL * L), group_body, 0)
        out_copies.append(
            pltpu.async_copy(
                resv.at[pl.ds(r * COLS, COLS)],
                out_hbm.at[r, pl.ds(col0, COLS)],
                osem,
            )
        )

    for oc in out_copies:
        oc.wait()


def kernel(x, exam_table, rel_table):
    xt = x.T
    exam = exam_table.T
    rel = rel_table.T
    out_t = _pbm_kernel(xt, exam, rel)
    return out_t.T
